# baseline (speedup 1.0000x reference)
import functools

import jax
import jax.numpy as jnp
from jax import lax
from jax.experimental import pallas as pl
from jax.experimental.pallas import tpu as pltpu
from jax.experimental.pallas import tpu_sc as plsc

N_SC = 2
N_TILE = 16
N_WORKER = N_SC * N_TILE
BLK_E = 128
BR = 2000

_MESH = dict(core_axis_name="c", subcore_axis_name="s",
             num_cores=N_SC, num_subcores=N_TILE)


def _wid(cid, sid):
    return sid * N_SC + cid


def _npad(n_nodes):
    return N_TILE * 128 * ((n_nodes + N_TILE * 128 - 1) // (N_TILE * 128))


@functools.lru_cache(maxsize=None)
def _make_deg(n_nodes, nb):
    npad = _npad(n_nodes)
    per_tile = npad // N_TILE

    def body(dst_ref, out_ref, dstv, onesv, zb, sdeg, sems):
        cid = lax.axis_index("c")
        sid = lax.axis_index("s")
        pltpu.async_copy(dst_ref.at[_wid(cid, sid)], dstv, sems)
        for i in range(per_tile // 16):
            zb[pl.ds(i * 16, 16)] = jnp.zeros((16,), jnp.float32)
        for i in range(BLK_E // 16):
            onesv[pl.ds(i * 16, 16)] = jnp.ones((16,), jnp.float32)
        pltpu.sync_copy(zb, sdeg.at[pl.ds(sid * per_tile, per_tile)])
        pltpu.make_async_copy(dst_ref.at[_wid(cid, sid)], dstv, sems).wait()
        plsc.subcore_barrier()

        def step(j, carry):
            pltpu.async_copy(onesv, sdeg.at[dstv.at[j]], sems, add=True)
            return carry

        lax.fori_loop(0, nb, step, 0)

        def wdrain(j, carry):
            pltpu.make_async_copy(onesv, sdeg.at[dstv.at[j]], sems).wait()
            return carry

        lax.fori_loop(0, nb, wdrain, 0)
        plsc.subcore_barrier()
        pltpu.sync_copy(sdeg.at[pl.ds(sid * per_tile, per_tile)],
                        out_ref.at[cid, pl.ds(sid * per_tile, per_tile)])

    return pl.kernel(
        body,
        out_type=jax.ShapeDtypeStruct((N_SC, npad), jnp.float32),
        mesh=plsc.VectorSubcoreMesh(**_MESH),
        scratch_types=[
            pltpu.VMEM((nb, BLK_E), jnp.int32),
            pltpu.VMEM((BLK_E,), jnp.float32),
            pltpu.VMEM((per_tile,), jnp.float32),
            pltpu.VMEM_SHARED((npad,), jnp.float32),
            pltpu.SemaphoreType.DMA,
        ],
    )


@functools.lru_cache(maxsize=None)
def _make_flow(n_nodes, d, nb):
    npad = _npad(n_nodes)
    rpt = npad // N_TILE
    assert rpt % BLK_E == 0 and nb % 2 == 1 and nb >= 3

    def body(src_ref, dst_ref, g_ref, out_ref,
             srcv, dsty, rows0, rows1, sacc, sem0, sem1, semd0, semd1, sems):
        cid = lax.axis_index("c")
        sid = lax.axis_index("s")
        w = _wid(cid, sid)
        pltpu.async_copy(src_ref.at[w], srcv, semd0)

        def zrow(r, carry):
            for c in range(d // 16):
                rows0[r, pl.ds(c * 16, 16)] = jnp.zeros((16,), jnp.float32)
            return carry

        lax.fori_loop(0, BLK_E, zrow, 0)
        for t in range(rpt // BLK_E):
            pltpu.async_copy(rows0,
                             sacc.at[pl.ds(sid * rpt + t * BLK_E, BLK_E)], sem0)
        for t in range(rpt // BLK_E):
            pltpu.make_async_copy(
                rows0, sacc.at[pl.ds(sid * rpt + t * BLK_E, BLK_E)],
                sem0).wait()
        pltpu.make_async_copy(src_ref.at[w], srcv, semd0).wait()

        def issue(j, rows, slot, semg, semd):
            pltpu.async_copy(g_ref.at[srcv.at[pl.ds(j * BLK_E, BLK_E)]],
                             rows, semg)
            pltpu.async_copy(dst_ref.at[w, j], dsty.at[slot], semd)

        def drain(j, rows, slot, semg, semd):
            pltpu.make_async_copy(
                g_ref.at[srcv.at[pl.ds(j * BLK_E, BLK_E)]], rows, semg).wait()
            pltpu.make_async_copy(dst_ref.at[w, j], dsty.at[slot], semd).wait()
            pltpu.sync_copy(rows, sacc.at[dsty.at[slot, 0]], add=True)

        issue(0, rows0, 0, sem0, semd0)
        issue(1, rows1, 1, sem1, semd1)
        plsc.subcore_barrier()

        def pair(p, carry):
            j0 = 2 * p
            drain(j0, rows0, 0, sem0, semd0)
            issue(j0 + 2, rows0, 0, sem0, semd0)
            drain(j0 + 1, rows1, 1, sem1, semd1)
            issue(j0 + 3, rows1, 1, sem1, semd1)
            return carry

        lax.fori_loop(0, (nb - 3) // 2, pair, 0)
        j = nb - 3
        drain(j, rows0, 0, sem0, semd0)
        issue(j + 2, rows0, 0, sem0, semd0)
        drain(j + 1, rows1, 1, sem1, semd1)
        drain(j + 2, rows0, 0, sem0, semd0)
        plsc.subcore_barrier()
        pltpu.sync_copy(sacc.at[pl.ds(sid * rpt, rpt)],
                        out_ref.at[cid, pl.ds(sid * rpt, rpt)])

    return pl.kernel(
        body,
        out_type=jax.ShapeDtypeStruct((N_SC, npad, d), jnp.float32),
        mesh=plsc.VectorSubcoreMesh(**_MESH),
        scratch_types=[
            pltpu.VMEM((nb * BLK_E,), jnp.int32),
            pltpu.VMEM((2, 1, BLK_E), jnp.int32),
            pltpu.VMEM((BLK_E, d), jnp.float32),
            pltpu.VMEM((BLK_E, d), jnp.float32),
            pltpu.VMEM_SHARED((npad, d), jnp.float32),
            pltpu.SemaphoreType.DMA,
            pltpu.SemaphoreType.DMA,
            pltpu.SemaphoreType.DMA,
            pltpu.SemaphoreType.DMA,
            pltpu.SemaphoreType.DMA,
        ],
    )


def _dinv_col(deg_ref):
    return lax.rsqrt(deg_ref[0] + deg_ref[1] + 1.0)


def _tc1_body(x_ref, w_ref, deg_ref, out_ref):
    out_ref[...] = jnp.dot(x_ref[...], w_ref[...],
                           preferred_element_type=jnp.float32) * _dinv_col(deg_ref)


def _tc2_body(p_ref, g_ref, deg_ref, w_ref, b_ref, out_ref):
    dinv = _dinv_col(deg_ref)
    z = jnp.maximum((p_ref[0] + p_ref[1] + g_ref[...]) * dinv + b_ref[...],
                    0.0)
    out_ref[...] = jnp.dot(z, w_ref[...],
                           preferred_element_type=jnp.float32) * dinv


def _tc3_body(p_ref, g_ref, deg_ref, b_ref, wfc_ref, bfc_ref, out_ref):
    z = jnp.maximum((p_ref[0] + p_ref[1] + g_ref[...]) * _dinv_col(deg_ref)
                    + b_ref[...], 0.0)
    out_ref[...] = jnp.dot(z, wfc_ref[...],
                           preferred_element_type=jnp.float32) + bfc_ref[...]


def kernel(x, edge_index, W1, b1, W2, b2, Wfc, bfc):
    n, d_in = x.shape
    d_hid = W1.shape[1]
    d_out = Wfc.shape[1]
    npad = _npad(n)
    ei = edge_index.astype(jnp.int32)
    src = ei[0].reshape(N_WORKER, -1)
    dst = ei[1].reshape(N_WORKER, -1)
    epw = src.shape[1]
    pad = (-epw) % BLK_E
    if pad:
        wcol = jnp.arange(N_WORKER, dtype=jnp.int32)[:, None]
        pcol = jnp.arange(pad, dtype=jnp.int32)[None, :]
        psrc = (wcol * 313 + pcol * 89) % n
        pdst = n + (wcol * 8 + pcol) % (npad - n)
        src = jnp.concatenate([src, psrc], axis=1)
        dst = jnp.concatenate([dst, pdst], axis=1)
    dst3 = dst.reshape(N_WORKER, -1, BLK_E)
    nb = dst3.shape[1]
    dst4 = dst.reshape(N_WORKER, nb, 1, BLK_E)

    degp = _make_deg(n, nb)(dst3)
    degT = degp.reshape(N_SC, -1, 1)
    flow = _make_flow(n, d_hid, nb)

    grid = (n // BR,)
    wspec = pl.BlockSpec((d_hid, d_hid), lambda i: (0, 0))
    dspec = pl.BlockSpec((N_SC, BR, 1), lambda i: (0, i, 0))
    rspec = pl.BlockSpec((BR, d_hid), lambda i: (i, 0))
    pspec = pl.BlockSpec((N_SC, BR, d_hid), lambda i: (0, i, 0))
    bspec = pl.BlockSpec((1, d_hid), lambda i: (0, 0))

    g1 = pl.pallas_call(
        _tc1_body,
        grid=grid,
        in_specs=[pl.BlockSpec((BR, d_in), lambda i: (i, 0)),
                  pl.BlockSpec((d_in, d_hid), lambda i: (0, 0)), dspec],
        out_specs=rspec,
        out_shape=jax.ShapeDtypeStruct((n, d_hid), jnp.float32),
    )(x, W1, degT)

    p1 = flow(src, dst4, g1)

    g2 = pl.pallas_call(
        _tc2_body,
        grid=grid,
        in_specs=[pspec, rspec, dspec, wspec, bspec],
        out_specs=rspec,
        out_shape=jax.ShapeDtypeStruct((n, d_hid), jnp.float32),
    )(p1, g1, degT, W2, b1.reshape(1, -1))

    p2 = flow(src, dst4, g2)

    out = pl.pallas_call(
        _tc3_body,
        grid=grid,
        in_specs=[pspec, rspec, dspec, bspec,
                  pl.BlockSpec((d_hid, d_out), lambda i: (0, 0)),
                  pl.BlockSpec((1, d_out), lambda i: (0, 0))],
        out_specs=pl.BlockSpec((BR, d_out), lambda i: (i, 0)),
        out_shape=jax.ShapeDtypeStruct((n, d_out), jnp.float32),
    )(p2, g2, degT, b2.reshape(1, -1), Wfc, bfc.reshape(1, -1))

    return out


# speedup vs baseline: 1.0262x; 1.0262x over previous
import functools

import jax
import jax.numpy as jnp
from jax import lax
from jax.experimental import pallas as pl
from jax.experimental.pallas import tpu as pltpu
from jax.experimental.pallas import tpu_sc as plsc

N_SC = 2
N_TILE = 16
N_WORKER = N_SC * N_TILE
BLK_E = 128
BR = 2000

_MESH = dict(core_axis_name="c", subcore_axis_name="s",
             num_cores=N_SC, num_subcores=N_TILE)


def _wid(cid, sid):
    return sid * N_SC + cid


def _npad(n_nodes):
    return N_TILE * 128 * ((n_nodes + N_TILE * 128 - 1) // (N_TILE * 128))


@functools.lru_cache(maxsize=None)
def _make_deg(n_nodes, nb):
    npad = _npad(n_nodes)
    per_tile = npad // N_TILE

    def body(dst_ref, out_ref, dstv, onesv, zb, sdeg, sems):
        cid = lax.axis_index("c")
        sid = lax.axis_index("s")
        pltpu.async_copy(dst_ref.at[_wid(cid, sid)], dstv, sems)
        for i in range(per_tile // 16):
            zb[pl.ds(i * 16, 16)] = jnp.zeros((16,), jnp.float32)
        for i in range(BLK_E // 16):
            onesv[pl.ds(i * 16, 16)] = jnp.ones((16,), jnp.float32)
        pltpu.sync_copy(zb, sdeg.at[pl.ds(sid * per_tile, per_tile)])
        pltpu.make_async_copy(dst_ref.at[_wid(cid, sid)], dstv, sems).wait()
        plsc.subcore_barrier()

        def step(j, carry):
            pltpu.async_copy(onesv, sdeg.at[dstv.at[j]], sems, add=True)
            return carry

        lax.fori_loop(0, nb, step, 0)

        def wdrain(j, carry):
            pltpu.make_async_copy(onesv, sdeg.at[dstv.at[j]], sems).wait()
            return carry

        lax.fori_loop(0, nb, wdrain, 0)
        plsc.subcore_barrier()
        pltpu.sync_copy(sdeg.at[pl.ds(sid * per_tile, per_tile)],
                        out_ref.at[cid, pl.ds(sid * per_tile, per_tile)])

    return pl.kernel(
        body,
        out_type=jax.ShapeDtypeStruct((N_SC, npad), jnp.float32),
        mesh=plsc.VectorSubcoreMesh(**_MESH),
        scratch_types=[
            pltpu.VMEM((nb, BLK_E), jnp.int32),
            pltpu.VMEM((BLK_E,), jnp.float32),
            pltpu.VMEM((per_tile,), jnp.float32),
            pltpu.VMEM_SHARED((npad,), jnp.float32),
            pltpu.SemaphoreType.DMA,
        ],
    )


@functools.lru_cache(maxsize=None)
def _make_flow(n_nodes, d, nb):
    npad = _npad(n_nodes)
    rpt = npad // N_TILE
    assert rpt % BLK_E == 0 and nb % 2 == 1 and nb >= 3

    def body(src_ref, dst_ref, g_ref, out_ref,
             srcv, dsty, rows0, rows1, sacc, sem0, sem1, semd0, semd1, sems):
        cid = lax.axis_index("c")
        sid = lax.axis_index("s")
        w = _wid(cid, sid)
        pltpu.async_copy(src_ref.at[w], srcv, semd0)

        def zrow(r, carry):
            for c in range(d // 16):
                rows0[r, pl.ds(c * 16, 16)] = jnp.zeros((16,), jnp.float32)
            return carry

        lax.fori_loop(0, BLK_E, zrow, 0)
        for t in range(rpt // BLK_E):
            pltpu.async_copy(rows0,
                             sacc.at[pl.ds(sid * rpt + t * BLK_E, BLK_E)], sem0)
        for t in range(rpt // BLK_E):
            pltpu.make_async_copy(
                rows0, sacc.at[pl.ds(sid * rpt + t * BLK_E, BLK_E)],
                sem0).wait()
        pltpu.make_async_copy(src_ref.at[w], srcv, semd0).wait()

        def issue(j, rows, slot, semg, semd):
            pltpu.async_copy(g_ref.at[srcv.at[pl.ds(j * BLK_E, BLK_E)]],
                             rows, semg)
            pltpu.async_copy(dst_ref.at[w, j], dsty.at[slot], semd)

        def drain(j, rows, slot, semg, semd):
            pltpu.make_async_copy(
                g_ref.at[srcv.at[pl.ds(j * BLK_E, BLK_E)]], rows, semg).wait()
            pltpu.make_async_copy(dst_ref.at[w, j], dsty.at[slot], semd).wait()
            pltpu.sync_copy(rows, sacc.at[dsty.at[slot, 0]], add=True)

        issue(0, rows0, 0, sem0, semd0)
        issue(1, rows1, 1, sem1, semd1)
        plsc.subcore_barrier()

        def pair(p, carry):
            j0 = 2 * p
            drain(j0, rows0, 0, sem0, semd0)
            issue(j0 + 2, rows0, 0, sem0, semd0)
            drain(j0 + 1, rows1, 1, sem1, semd1)
            issue(j0 + 3, rows1, 1, sem1, semd1)
            return carry

        lax.fori_loop(0, (nb - 3) // 2, pair, 0)
        j = nb - 3
        drain(j, rows0, 0, sem0, semd0)
        issue(j + 2, rows0, 0, sem0, semd0)
        drain(j + 1, rows1, 1, sem1, semd1)
        drain(j + 2, rows0, 0, sem0, semd0)
        plsc.subcore_barrier()
        pltpu.sync_copy(sacc.at[pl.ds(sid * rpt, rpt)],
                        out_ref.at[cid, pl.ds(sid * rpt, rpt)])

    return pl.kernel(
        body,
        out_type=jax.ShapeDtypeStruct((N_SC, npad, d), jnp.float32),
        mesh=plsc.VectorSubcoreMesh(**_MESH),
        scratch_types=[
            pltpu.VMEM((nb * BLK_E,), jnp.int32),
            pltpu.VMEM((2, 1, BLK_E), jnp.int32),
            pltpu.VMEM((BLK_E, d), jnp.float32),
            pltpu.VMEM((BLK_E, d), jnp.float32),
            pltpu.VMEM_SHARED((npad, d), jnp.float32),
            pltpu.SemaphoreType.DMA,
            pltpu.SemaphoreType.DMA,
            pltpu.SemaphoreType.DMA,
            pltpu.SemaphoreType.DMA,
            pltpu.SemaphoreType.DMA,
        ],
    )


def _dinv_col(deg_ref):
    return lax.rsqrt(deg_ref[0] + deg_ref[1] + 1.0)


def _tc1_body(x_ref, w_ref, deg_ref, out_ref):
    out_ref[...] = jnp.dot(x_ref[...], w_ref[...],
                           preferred_element_type=jnp.float32) * _dinv_col(deg_ref)


def _tc2_body(p_ref, g_ref, deg_ref, w_ref, b_ref, out_ref):
    dinv = _dinv_col(deg_ref)
    z = jnp.maximum((p_ref[0] + p_ref[1] + g_ref[...]) * dinv + b_ref[...],
                    0.0)
    out_ref[...] = jnp.dot(z, w_ref[...],
                           preferred_element_type=jnp.float32) * dinv


def _tc3_body(p_ref, g_ref, deg_ref, b_ref, wfc_ref, bfc_ref, out_ref):
    z = jnp.maximum((p_ref[0] + p_ref[1] + g_ref[...]) * _dinv_col(deg_ref)
                    + b_ref[...], 0.0)
    out_ref[...] = jnp.dot(z, wfc_ref[...],
                           preferred_element_type=jnp.float32) + bfc_ref[...]


def kernel(x, edge_index, W1, b1, W2, b2, Wfc, bfc):
    n, d_in = x.shape
    d_hid = W1.shape[1]
    d_out = Wfc.shape[1]
    npad = _npad(n)
    ei = edge_index.astype(jnp.int32)
    ne = ei.shape[1]
    eb = ne // 10

    def _split_body(ei_ref, s_ref, d_ref):
        s_ref[...] = ei_ref[0]
        d_ref[...] = ei_ref[1]

    src_f, dst_f = pl.pallas_call(
        _split_body,
        out_shape=(jax.ShapeDtypeStruct((ne,), jnp.int32),
                   jax.ShapeDtypeStruct((ne,), jnp.int32)),
    )(ei)
    src = src_f.reshape(N_WORKER, -1)
    dst = dst_f.reshape(N_WORKER, -1)
    epw = src.shape[1]
    pad = (-epw) % BLK_E
    if pad:
        wcol = jnp.arange(N_WORKER, dtype=jnp.int32)[:, None]
        pcol = jnp.arange(pad, dtype=jnp.int32)[None, :]
        psrc = (wcol * 313 + pcol * 89) % n
        pdst = n + (wcol * 8 + pcol) % (npad - n)
        src = jnp.concatenate([src, psrc], axis=1)
        dst = jnp.concatenate([dst, pdst], axis=1)
    dst3 = dst.reshape(N_WORKER, -1, BLK_E)
    nb = dst3.shape[1]
    dst4 = dst.reshape(N_WORKER, nb, 1, BLK_E)

    degp = _make_deg(n, nb)(dst3)
    degT = degp.reshape(N_SC, -1, 1)
    flow = _make_flow(n, d_hid, nb)

    grid = (n // BR,)
    wspec = pl.BlockSpec((d_hid, d_hid), lambda i: (0, 0))
    dspec = pl.BlockSpec((N_SC, BR, 1), lambda i: (0, i, 0))
    rspec = pl.BlockSpec((BR, d_hid), lambda i: (i, 0))
    pspec = pl.BlockSpec((N_SC, BR, d_hid), lambda i: (0, i, 0))
    bspec = pl.BlockSpec((1, d_hid), lambda i: (0, 0))

    g1 = pl.pallas_call(
        _tc1_body,
        grid=grid,
        in_specs=[pl.BlockSpec((BR, d_in), lambda i: (i, 0)),
                  pl.BlockSpec((d_in, d_hid), lambda i: (0, 0)), dspec],
        out_specs=rspec,
        out_shape=jax.ShapeDtypeStruct((n, d_hid), jnp.float32),
    )(x, W1, degT)

    p1 = flow(src, dst4, g1)

    g2 = pl.pallas_call(
        _tc2_body,
        grid=grid,
        in_specs=[pspec, rspec, dspec, wspec, bspec],
        out_specs=rspec,
        out_shape=jax.ShapeDtypeStruct((n, d_hid), jnp.float32),
    )(p1, g1, degT, W2, b1.reshape(1, -1))

    p2 = flow(src, dst4, g2)

    out = pl.pallas_call(
        _tc3_body,
        grid=grid,
        in_specs=[pspec, rspec, dspec, bspec,
                  pl.BlockSpec((d_hid, d_out), lambda i: (0, 0)),
                  pl.BlockSpec((1, d_out), lambda i: (0, 0))],
        out_specs=pl.BlockSpec((BR, d_out), lambda i: (i, 0)),
        out_shape=jax.ShapeDtypeStruct((n, d_out), jnp.float32),
    )(p2, g2, degT, b2.reshape(1, -1), Wfc, bfc.reshape(1, -1))

    return out
